# Initial kernel scaffold; baseline (speedup 1.0000x reference)
import jax
import jax.numpy as jnp
from jax.experimental import pallas as pl


def kernel(x, freqs_cis, mask, attn_norm_w, ffn_norm_w, wq, wk, wv, wo, gate_w, w1e, w2e, w3e, w1s, w2s, w3s):
    raise NotImplementedError("write your pallas kernel here")


# speedup vs baseline: 1.3075x; 1.3075x over previous
import functools
import math

import jax
import jax.numpy as jnp
from jax.experimental import pallas as pl

DIM = 4096
NHEADS = 32
SEQLEN = 128
HEAD_DIM = DIM // NHEADS
NEXP = 64
TOPK = 6
NGROUPS = 8
GSIZE = NEXP // NGROUPS
TOPKG = 4
MOE_INTER = 1024
SHARED_INTER = 2 * MOE_INTER
ROUTE_SCALE = 2.5
EPS = 1e-5
NEG = -1e30

MOE_BLK = 512
SHARED_BLK = 512


def _bdot(a, b):
    return jnp.dot(a.astype(jnp.bfloat16), b.astype(jnp.bfloat16),
                   preferred_element_type=jnp.float32)

def _rmsnorm(x, w):
    return x * jax.lax.rsqrt(jnp.mean(x * x, axis=-1, keepdims=True) + EPS) * w


def _attn_head_kernel(x_ref, nw_ref, wq_ref, wk_ref, wv_ref, f_ref, m_ref, o_ref):
    xn = _rmsnorm(x_ref[...], nw_ref[...])
    f = f_ref[...]
    q = _bdot(xn, wq_ref[...].T) * f
    k = _bdot(xn, wk_ref[...].T) * f
    v = _bdot(xn, wv_ref[...].T)
    s = _bdot(q, k.T)
    s = s * (1.0 / math.sqrt(HEAD_DIM)) + m_ref[...]
    s = s - jnp.max(s, axis=-1, keepdims=True)
    p = jnp.exp(s)
    p = p / jnp.sum(p, axis=-1, keepdims=True)
    o_ref[...] = _bdot(p, v)


def _attn_out_kernel(a_ref, wo_ref, x_ref, o_ref):
    o_ref[...] = x_ref[...] + _bdot(a_ref[...], wo_ref[...].T)


def _gate_kernel(h_ref, nw_ref, gw_ref, xn_ref, comb_ref):
    xn = _rmsnorm(h_ref[...], nw_ref[...])
    xn_ref[...] = xn
    s = jax.nn.sigmoid(_bdot(xn, gw_ref[...].T))
    T = s.shape[0]
    col = jax.lax.broadcasted_iota(jnp.int32, (T, NEXP), 1)
    grp = col // GSIZE

    gmax_full = jnp.zeros_like(s)
    for j in range(NGROUPS):
        mj = jnp.max(jnp.where(grp == j, s, NEG), axis=1, keepdims=True)
        gmax_full = jnp.where(grp == j, mj, gmax_full)

    rep = jnp.where(col % GSIZE == 0, gmax_full, NEG)
    kept_rep = jnp.zeros_like(s)
    cur = rep
    for _ in range(TOPKG):
        mx = jnp.max(cur, axis=1, keepdims=True)
        mincol = jnp.min(jnp.where(cur == mx, col, NEXP), axis=1, keepdims=True)
        first = jnp.where((cur == mx) & (col == mincol), 1.0, 0.0)
        kept_rep = kept_rep + first
        cur = jnp.where(first > 0.5, NEG, cur)
    kept_full = jnp.zeros_like(s)
    for j in range(NGROUPS):
        sel = kept_rep[:, j * GSIZE:j * GSIZE + 1]
        kept_full = jnp.where(grp == j, jnp.broadcast_to(sel, (T, NEXP)),
                              kept_full)

    sel_mask = jnp.zeros_like(s)
    cur = jnp.where(kept_full > 0.5, s, NEG)
    for _ in range(TOPK):
        mx = jnp.max(cur, axis=1, keepdims=True)
        mincol = jnp.min(jnp.where(cur == mx, col, NEXP), axis=1, keepdims=True)
        first = jnp.where((cur == mx) & (col == mincol), 1.0, 0.0)
        sel_mask = sel_mask + first
        cur = jnp.where(first > 0.5, NEG, cur)

    w = s * sel_mask
    comb_ref[...] = w / jnp.sum(w, axis=1, keepdims=True) * ROUTE_SCALE


def _moe_kernel(xn_ref, comb_ref, w1_ref, w3_ref, w2_ref, o_ref):
    e = pl.program_id(0)
    j = pl.program_id(1)
    xn = xn_ref[...]
    h1 = _bdot(xn, w1_ref[0].T)
    h3 = _bdot(xn, w3_ref[0].T)
    act = jax.nn.silu(h1) * h3
    part = _bdot(act, w2_ref[0].T)
    comb = comb_ref[...]
    col = jax.lax.broadcasted_iota(jnp.int32, comb.shape, 1)
    scale = jnp.sum(jnp.where(col == e, comb, 0.0), axis=1, keepdims=True)
    part = part * scale

    @pl.when(jnp.logical_and(e == 0, j == 0))
    def _():
        o_ref[...] = part

    @pl.when(jnp.logical_or(e != 0, j != 0))
    def _():
        o_ref[...] += part


def _shared_kernel(xn_ref, h_ref, y_ref, w1_ref, w3_ref, w2_ref, o_ref):
    j = pl.program_id(0)
    xn = xn_ref[...]
    h1 = _bdot(xn, w1_ref[...].T)
    h3 = _bdot(xn, w3_ref[...].T)
    act = jax.nn.silu(h1) * h3
    part = _bdot(act, w2_ref[...].T)

    @pl.when(j == 0)
    def _():
        o_ref[...] = h_ref[...] + y_ref[...] + part

    @pl.when(j != 0)
    def _():
        o_ref[...] += part


def kernel(x, freqs_cis, mask, attn_norm_w, ffn_norm_w, wq, wk, wv, wo,
           gate_w, w1e, w2e, w3e, w1s, w2s, w3s, interpret=False):
    x2 = x.reshape(SEQLEN, DIM)
    f_full = jnp.repeat(freqs_cis, 2, axis=1)
    mask2 = mask.reshape(SEQLEN, SEQLEN)
    anw = attn_norm_w.reshape(1, DIM)
    fnw = ffn_norm_w.reshape(1, DIM)

    T, D, H = SEQLEN, DIM, HEAD_DIM

    attn = pl.pallas_call(
        _attn_head_kernel,
        grid=(NHEADS,),
        in_specs=[
            pl.BlockSpec((T, D), lambda h: (0, 0)),
            pl.BlockSpec((1, D), lambda h: (0, 0)),
            pl.BlockSpec((H, D), lambda h: (h, 0)),
            pl.BlockSpec((H, D), lambda h: (h, 0)),
            pl.BlockSpec((H, D), lambda h: (h, 0)),
            pl.BlockSpec((T, H), lambda h: (0, 0)),
            pl.BlockSpec((T, T), lambda h: (0, 0)),
        ],
        out_specs=pl.BlockSpec((T, H), lambda h: (0, h)),
        out_shape=jax.ShapeDtypeStruct((T, D), jnp.float32),
        interpret=interpret,
    )(x2, anw, wq, wk, wv, f_full, mask2)

    OB = 512
    h = pl.pallas_call(
        _attn_out_kernel,
        grid=(D // OB,),
        in_specs=[
            pl.BlockSpec((T, D), lambda j: (0, 0)),
            pl.BlockSpec((OB, D), lambda j: (j, 0)),
            pl.BlockSpec((T, OB), lambda j: (0, j)),
        ],
        out_specs=pl.BlockSpec((T, OB), lambda j: (0, j)),
        out_shape=jax.ShapeDtypeStruct((T, D), jnp.float32),
        interpret=interpret,
    )(attn, wo, x2)

    xn2, comb = pl.pallas_call(
        _gate_kernel,
        grid=(1,),
        in_specs=[
            pl.BlockSpec((T, D), lambda i: (0, 0)),
            pl.BlockSpec((1, D), lambda i: (0, 0)),
            pl.BlockSpec((NEXP, D), lambda i: (0, 0)),
        ],
        out_specs=[
            pl.BlockSpec((T, D), lambda i: (0, 0)),
            pl.BlockSpec((T, NEXP), lambda i: (0, 0)),
        ],
        out_shape=[
            jax.ShapeDtypeStruct((T, D), jnp.float32),
            jax.ShapeDtypeStruct((T, NEXP), jnp.float32),
        ],
        interpret=interpret,
    )(h, fnw, gate_w)

    B = MOE_BLK
    y = pl.pallas_call(
        _moe_kernel,
        grid=(NEXP, MOE_INTER // B),
        in_specs=[
            pl.BlockSpec((T, D), lambda e, j: (0, 0)),
            pl.BlockSpec((T, NEXP), lambda e, j: (0, 0)),
            pl.BlockSpec((1, B, D), lambda e, j: (e, j, 0)),
            pl.BlockSpec((1, B, D), lambda e, j: (e, j, 0)),
            pl.BlockSpec((1, D, B), lambda e, j: (e, 0, j)),
        ],
        out_specs=pl.BlockSpec((T, D), lambda e, j: (0, 0)),
        out_shape=jax.ShapeDtypeStruct((T, D), jnp.float32),
        interpret=interpret,
    )(xn2, comb, w1e, w3e, w2e)

    SB = SHARED_BLK
    out = pl.pallas_call(
        _shared_kernel,
        grid=(SHARED_INTER // SB,),
        in_specs=[
            pl.BlockSpec((T, D), lambda j: (0, 0)),
            pl.BlockSpec((T, D), lambda j: (0, 0)),
            pl.BlockSpec((T, D), lambda j: (0, 0)),
            pl.BlockSpec((SB, D), lambda j: (j, 0)),
            pl.BlockSpec((SB, D), lambda j: (j, 0)),
            pl.BlockSpec((D, SB), lambda j: (0, j)),
        ],
        out_specs=pl.BlockSpec((T, D), lambda j: (0, 0)),
        out_shape=jax.ShapeDtypeStruct((T, D), jnp.float32),
        interpret=interpret,
    )(xn2, h, y, w1s, w3s, w2s)

    return out.reshape(1, SEQLEN, DIM)
